# baseline (speedup 1.0000x reference)
import jax
import jax.numpy as jnp
from jax.experimental import pallas as pl
from jax.experimental.pallas import tpu as pltpu

TOTAL = 16384
D = 2048
SEG = 2048
NSEG = TOTAL // SEG
BM = 1024
QROWS = 256
QPH = BM // QROWS
NQ = TOTAL // QROWS
NS = 12
NSTEP = TOTAL // BM + 2

_NT = (((1,), (1,)), ((), ()))


def _body(x_hbm, w_ref, b_ref, o_ref, ring, xc_ref, sum_ref, xm_ref,
          sems):
    t = pl.program_id(0)

    def qcopy(qi):
        return pltpu.make_async_copy(
            x_hbm.at[pl.ds(qi * QROWS, QROWS), :], ring.at[qi % NS],
            sems.at[qi % NS])

    @pl.when(t == 0)
    def _():
        for k in range(QPH):
            qcopy(k).start()

    for k in range(QPH):
        @pl.when(QPH * (t + 1) + k < NQ)
        def _(k=k):
            qcopy(QPH * (t + 1) + k).start()

    @pl.when(t < NSTEP - 2)
    def _():
        for k in range(QPH):
            qcopy(QPH * t + k).wait()

    q0 = QPH * t
    cs = jnp.zeros((1, D), jnp.float32)
    for k in range(QPH):
        cs = cs + jnp.sum(ring[jnp.minimum(q0 + k, NQ - 1) % NS],
                          axis=0, keepdims=True)

    @pl.when(t % 2 == 0)
    def _():
        sum_ref[...] = cs

    @pl.when(t % 2 == 1)
    def _():
        xm_ref[...] = (sum_ref[...] + cs) * (1.0 / SEG)

    p0 = QPH * (t - 1)
    xm = xm_ref[...]
    for k in range(QPH):
        xc_ref[(t + 1) % 2, pl.ds(k * QROWS, QROWS), :] = (
            ring[jnp.clip(p0 + k, 0, NQ - 1) % NS] - xm
        ).astype(jnp.bfloat16)

    o_ref[...] = jax.lax.dot_general(
        xc_ref[t % 2], w_ref[...], dimension_numbers=_NT,
        preferred_element_type=jnp.float32) + b_ref[...]


def kernel(x, W, b, l):
    b_eff = (b + (jnp.asarray(l) - SEG).astype(jnp.float32)).reshape(1, D)
    W_bf = W.astype(jnp.bfloat16)

    out = pl.pallas_call(
        _body,
        grid=(NSTEP,),
        in_specs=[
            pl.BlockSpec(memory_space=pltpu.MemorySpace.HBM),
            pl.BlockSpec((D, D), lambda t: (0, 0)),
            pl.BlockSpec((1, D), lambda t: (0, 0)),
        ],
        out_specs=pl.BlockSpec(
            (BM, D), lambda t: (jnp.maximum(t - 2, 0), 0)),
        out_shape=jax.ShapeDtypeStruct((TOTAL, D), jnp.float32),
        scratch_shapes=[
            pltpu.VMEM((NS, QROWS, D), jnp.float32),
            pltpu.VMEM((2, BM, D), jnp.bfloat16),
            pltpu.VMEM((1, D), jnp.float32),
            pltpu.VMEM((1, D), jnp.float32),
            pltpu.SemaphoreType.DMA((NS,)),
        ],
        compiler_params=pltpu.CompilerParams(
            vmem_limit_bytes=64 * 1024 * 1024,
        ),
    )(x, W_bf, b_eff)
    return out


# speedup vs baseline: 1.1205x; 1.1205x over previous
import jax
import jax.numpy as jnp
from jax.experimental import pallas as pl
from jax.experimental.pallas import tpu as pltpu

TOTAL = 16384
D = 2048
SEG = 2048
NSEG = TOTAL // SEG
BM = 1024
M_TILES = SEG // BM


def _fused_body(x_ref, w_ref, b_ref, o_ref, xm_ref):
    m = pl.program_id(1)

    @pl.when(m == 0)
    def _():
        xm_ref[...] = jnp.mean(x_ref[...], axis=0, keepdims=True)

    xc = (x_ref[pl.ds(m * BM, BM), :] - xm_ref[...]).astype(jnp.bfloat16)
    o_ref[...] = jax.lax.dot_general(
        xc, w_ref[...],
        dimension_numbers=(((1,), (0,)), ((), ())),
        preferred_element_type=jnp.float32,
    ) + b_ref[...]


def kernel(x, W, b, l):
    b_eff = (b + (jnp.asarray(l) - SEG).astype(jnp.float32)).reshape(1, D)
    Wt_bf = W.T.astype(jnp.bfloat16)

    out = pl.pallas_call(
        _fused_body,
        grid=(NSEG, M_TILES),
        in_specs=[
            pl.BlockSpec((SEG, D), lambda s, m: (s, 0)),
            pl.BlockSpec((D, D), lambda s, m: (0, 0)),
            pl.BlockSpec((1, D), lambda s, m: (0, 0)),
        ],
        out_specs=pl.BlockSpec((BM, D), lambda s, m: (s * M_TILES + m, 0)),
        out_shape=jax.ShapeDtypeStruct((TOTAL, D), jnp.float32),
        scratch_shapes=[pltpu.VMEM((1, D), jnp.float32)],
        compiler_params=pltpu.CompilerParams(
            vmem_limit_bytes=64 * 1024 * 1024,
        ),
    )(x, Wt_bf, b_eff)
    return out
